# baseline (speedup 1.0000x reference)
import functools

import jax
import jax.numpy as jnp
from jax import lax
from jax.experimental import pallas as pl
from jax.experimental.pallas import tpu as pltpu
from jax.experimental.pallas import tpu_sc as plsc

B, T, N = 32, 12, 1024
INPUT_DIM = 3
IN_EMB, TOD_EMB, DOW_EMB, ADP_EMB = 24, 24, 24, 80
STEPS_PER_DAY = 288
OUT_DIM = IN_EMB + TOD_EMB + DOW_EMB + ADP_EMB
ROWS = B * T * N
NW = 32
RPW = ROWS // NW
CHUNK = 256
NCHUNK = RPW // CHUNK
L = 16
ADP_OFF = IN_EMB + TOD_EMB + DOW_EMB

DOW_ROW = 56
WB_ROW = 64
X_ROW = 72
ADP_ROW = 112
ARENA_ROWS = 272

DOW_BASE = DOW_ROW * 128
WB_BASE = WB_ROW * 128
X_BASE = X_ROW * 128
ADP_BASE = ADP_ROW * 128


def _sc_body(x_hbm, wb_hbm, tod_hbm, dow_hbm, adp_hbm, out_hbm, arena, stage):
    c = lax.axis_index("c")
    s = lax.axis_index("s")
    wid = s * 2 + c
    base = wid * RPW

    def ag(flat_idx):
        return plsc.load_gather(arena, [flat_idx >> 7, flat_idx & 127])

    pltpu.sync_copy(tod_hbm, arena.at[pl.ds(0, 54), :])
    pltpu.sync_copy(dow_hbm, arena.at[pl.ds(DOW_ROW, 2), :])
    pltpu.sync_copy(wb_hbm, arena.at[pl.ds(WB_ROW, 8), :])

    lanes = lax.iota(jnp.int32, L)
    zeros = jnp.zeros((L,), jnp.int32)
    wv = [ag(lanes + (WB_BASE + k * 32 + o))
          for k in range(INPUT_DIM) for o in (0, L)]
    bv = [ag(lanes + (WB_BASE + INPUT_DIM * 32 + o)) for o in (0, L)]

    def chunk_body(ck, _):
        l0 = ck * CHUNK

        @pl.when(ck % 4 == 0)
        def _load_x():
            x_off = pl.multiple_of((base + l0) * 5 // 128, 40)
            pltpu.sync_copy(x_hbm.at[pl.ds(x_off, 40), :],
                            arena.at[pl.ds(X_ROW, 40), :])

        a_off = pl.multiple_of(l0 * ADP_EMB // 128, CHUNK * ADP_EMB // 128)
        pltpu.sync_copy(adp_hbm.at[pl.ds(a_off, CHUNK * ADP_EMB // 128), :],
                        arena.at[pl.ds(ADP_ROW, CHUNK * ADP_EMB // 128), :])

        xw = X_BASE + (ck % 4) * (CHUNK * 5)

        def group_body(g, _):
            rid = lanes + g * L
            rid5 = rid * 5 + xw
            x0 = ag(rid5)
            x1 = ag(rid5 + 1)
            x2 = ag(rid5 + 2)
            x3 = ag(rid5 + 3)
            x4 = ag(rid5 + 4)
            tod_i = jnp.clip((x3 * STEPS_PER_DAY).astype(jnp.int32),
                             0, STEPS_PER_DAY - 1)
            dow_i = jnp.clip(x4.astype(jnp.int32), 0, 6)
            t24 = tod_i * TOD_EMB
            d24 = dow_i * DOW_EMB + DOW_BASE
            r80 = rid * ADP_EMB + ADP_BASE
            for j in range(IN_EMB):
                jv = zeros + j
                tv = ag(t24 + j)
                plsc.store_scatter(stage, [rid, jv + IN_EMB], tv)
                dv = ag(d24 + j)
                plsc.store_scatter(stage, [rid, jv + (IN_EMB + TOD_EMB)], dv)
                hi, lo = j // L, j % L
                hj = (x0 * wv[2 * 0 + hi][lo] + x1 * wv[2 * 1 + hi][lo]
                      + x2 * wv[2 * 2 + hi][lo] + bv[hi][lo])
                plsc.store_scatter(stage, [rid, jv], hj)
            for j in range(ADP_EMB):
                av = ag(r80 + j)
                plsc.store_scatter(stage, [rid, (zeros + j) + ADP_OFF], av)
            return 0

        lax.fori_loop(0, CHUNK // L, group_body, 0)
        o_off = pl.multiple_of(base + l0, CHUNK)
        pltpu.sync_copy(stage, out_hbm.at[pl.ds(o_off, CHUNK), :])
        return 0

    lax.fori_loop(0, NCHUNK, chunk_body, 0)


@functools.partial(jax.jit, static_argnames=("interpret",))
def kernel(x, W_proj, b_proj, tod_table, dow_table, adaptive, interpret=False):
    del interpret
    x_flat = x.reshape(ROWS * 5 // 128, 128)
    adp_flat = adaptive.reshape(T * N * ADP_EMB // 128, 128)
    tod_flat = jnp.concatenate(
        [tod_table.reshape(STEPS_PER_DAY * TOD_EMB),
         jnp.zeros((54 * 128 - STEPS_PER_DAY * TOD_EMB,), jnp.float32)]
    ).reshape(54, 128)
    dow_flat = jnp.concatenate(
        [dow_table.reshape(7 * DOW_EMB),
         jnp.zeros((2 * 128 - 7 * DOW_EMB,), jnp.float32)]
    ).reshape(2, 128)
    wb_flat = jnp.zeros((8, 128), jnp.float32)
    for k in range(INPUT_DIM):
        wb_flat = wb_flat.at[0, k * 32:k * 32 + IN_EMB].set(W_proj[k])
    wb_flat = wb_flat.at[0, INPUT_DIM * 32:INPUT_DIM * 32 + IN_EMB].set(b_proj)
    mesh = plsc.VectorSubcoreMesh(core_axis_name="c", subcore_axis_name="s")
    run = pl.kernel(
        _sc_body,
        mesh=mesh,
        compiler_params=pltpu.CompilerParams(needs_layout_passes=False),
        out_type=jax.ShapeDtypeStruct((ROWS, OUT_DIM), jnp.float32),
        scratch_types=[
            pltpu.VMEM((ARENA_ROWS, 128), jnp.float32),
            pltpu.VMEM((CHUNK, OUT_DIM), jnp.float32),
        ],
    )
    out = run(x_flat, wb_flat, tod_flat, dow_flat, adp_flat)
    return out.reshape(B, T, N, OUT_DIM)


# speedup vs baseline: 2.8523x; 2.8523x over previous
import functools

import jax
import jax.numpy as jnp
from jax import lax
from jax.experimental import pallas as pl
from jax.experimental.pallas import tpu as pltpu

B, T, N = 32, 12, 1024
INPUT_DIM = 3
IN_EMB, TOD_EMB, DOW_EMB, ADP_EMB = 24, 24, 24, 80
STEPS_PER_DAY = 288
OUT_DIM = IN_EMB + TOD_EMB + DOW_EMB + ADP_EMB
K_OH = 304
BB = 4


def _body(x_ref, w_ref, b_ref, tcat_ref, adp_ref, out_ref):
    M = BB * N
    xb = x_ref[:, 0].reshape(M, INPUT_DIM + 2)
    xi = xb[:, :INPUT_DIM]
    h = jnp.dot(xi, w_ref[...], preferred_element_type=jnp.float32) + b_ref[0]

    tod_idx = (xb[:, INPUT_DIM] * STEPS_PER_DAY).astype(jnp.int32)
    tod_idx = jnp.clip(tod_idx, 0, STEPS_PER_DAY - 1)
    dow_idx = xb[:, INPUT_DIM + 1].astype(jnp.int32)
    dow_idx = jnp.clip(dow_idx, 0, 6)

    lane = lax.broadcasted_iota(jnp.int16, (M, K_OH), 1)
    tod16 = tod_idx.astype(jnp.int16)
    dow16 = (dow_idx + STEPS_PER_DAY).astype(jnp.int16)
    oh = ((lane == tod16[:, None])
          | (lane == dow16[:, None])).astype(jnp.bfloat16)
    emb = jnp.dot(oh, tcat_ref[...], preferred_element_type=jnp.float32)

    h4 = h.reshape(BB, N, IN_EMB)
    e4 = emb.reshape(BB, N, OUT_DIM)
    for i in range(BB):
        out_ref[i, 0] = e4[i]
        out_ref[i, 0, :, 0:IN_EMB] = h4[i]
        out_ref[i, 0, :, IN_EMB + TOD_EMB + DOW_EMB:OUT_DIM] = adp_ref[0]


@functools.partial(jax.jit, static_argnames=("interpret",))
def kernel(x, W_proj, b_proj, tod_table, dow_table, adaptive, interpret=False):
    tcat = jnp.zeros((K_OH, OUT_DIM), jnp.bfloat16)
    tcat = tcat.at[:STEPS_PER_DAY, IN_EMB:IN_EMB + TOD_EMB].set(
        tod_table.astype(jnp.bfloat16))
    tcat = tcat.at[STEPS_PER_DAY:STEPS_PER_DAY + 7,
                   IN_EMB + TOD_EMB:IN_EMB + TOD_EMB + DOW_EMB].set(
        dow_table.astype(jnp.bfloat16))
    grid = (T, B // BB)
    return pl.pallas_call(
        _body,
        grid=grid,
        in_specs=[
            pl.BlockSpec((BB, 1, N, INPUT_DIM + 2), lambda t, b: (b, t, 0, 0)),
            pl.BlockSpec((INPUT_DIM, IN_EMB), lambda t, b: (0, 0)),
            pl.BlockSpec((1, IN_EMB), lambda t, b: (0, 0)),
            pl.BlockSpec((K_OH, OUT_DIM), lambda t, b: (0, 0)),
            pl.BlockSpec((1, N, ADP_EMB), lambda t, b: (t, 0, 0)),
        ],
        out_specs=pl.BlockSpec((BB, 1, N, OUT_DIM), lambda t, b: (b, t, 0, 0)),
        out_shape=jax.ShapeDtypeStruct((B, T, N, OUT_DIM), jnp.float32),
        compiler_params=pltpu.CompilerParams(
            dimension_semantics=("arbitrary", "arbitrary"),
        ),
        interpret=interpret,
    )(x, W_proj, b_proj.reshape(1, IN_EMB), tcat, adaptive)


# speedup vs baseline: 2.9679x; 1.0405x over previous
import functools

import jax
import jax.numpy as jnp
from jax import lax
from jax.experimental import pallas as pl
from jax.experimental.pallas import tpu as pltpu

B, T, N = 32, 12, 1024
INPUT_DIM = 3
IN_EMB, TOD_EMB, DOW_EMB, ADP_EMB = 24, 24, 24, 80
STEPS_PER_DAY = 288
OUT_DIM = IN_EMB + TOD_EMB + DOW_EMB + ADP_EMB
K_OH = 304
BB = 4


def _body(x_ref, w_ref, b_ref, tcat_ref, adp_ref, out_ref):
    M = BB * N
    xb = x_ref[:, 0].reshape(M, INPUT_DIM + 2)
    xi = xb[:, :INPUT_DIM]
    h = jnp.dot(xi, w_ref[...], preferred_element_type=jnp.float32) + b_ref[0]

    tod_idx = (xb[:, INPUT_DIM] * STEPS_PER_DAY).astype(jnp.int32)
    tod_idx = jnp.clip(tod_idx, 0, STEPS_PER_DAY - 1)
    dow_idx = xb[:, INPUT_DIM + 1].astype(jnp.int32)
    dow_idx = jnp.clip(dow_idx, 0, 6)

    lane = lax.broadcasted_iota(jnp.int32, (M, K_OH), 1)
    oh = ((lane == tod_idx[:, None])
          | (lane == dow_idx[:, None] + STEPS_PER_DAY)).astype(jnp.bfloat16)
    emb = jnp.dot(oh, tcat_ref[...], preferred_element_type=jnp.float32)

    h4 = h.reshape(BB, N, IN_EMB)
    e4 = emb.reshape(BB, N, OUT_DIM)
    for i in range(BB):
        out_ref[i, 0] = e4[i]
        out_ref[i, 0, :, 0:IN_EMB] = h4[i]
        out_ref[i, 0, :, IN_EMB + TOD_EMB + DOW_EMB:OUT_DIM] = adp_ref[0]


@functools.partial(jax.jit, static_argnames=("interpret",))
def kernel(x, W_proj, b_proj, tod_table, dow_table, adaptive, interpret=False):
    tcat = jnp.zeros((K_OH, OUT_DIM), jnp.bfloat16)
    tcat = tcat.at[:STEPS_PER_DAY, IN_EMB:IN_EMB + TOD_EMB].set(
        tod_table.astype(jnp.bfloat16))
    tcat = tcat.at[STEPS_PER_DAY:STEPS_PER_DAY + 7,
                   IN_EMB + TOD_EMB:IN_EMB + TOD_EMB + DOW_EMB].set(
        dow_table.astype(jnp.bfloat16))
    grid = (T, B // BB)
    return pl.pallas_call(
        _body,
        grid=grid,
        in_specs=[
            pl.BlockSpec((BB, 1, N, INPUT_DIM + 2), lambda t, b: (b, t, 0, 0)),
            pl.BlockSpec((INPUT_DIM, IN_EMB), lambda t, b: (0, 0)),
            pl.BlockSpec((1, IN_EMB), lambda t, b: (0, 0)),
            pl.BlockSpec((K_OH, OUT_DIM), lambda t, b: (0, 0)),
            pl.BlockSpec((1, N, ADP_EMB), lambda t, b: (t, 0, 0)),
        ],
        out_specs=pl.BlockSpec((BB, 1, N, OUT_DIM), lambda t, b: (b, t, 0, 0)),
        out_shape=jax.ShapeDtypeStruct((B, T, N, OUT_DIM), jnp.float32),
        compiler_params=pltpu.CompilerParams(
            dimension_semantics=("parallel", "parallel"),
        ),
        interpret=interpret,
    )(x, W_proj, b_proj.reshape(1, IN_EMB), tcat, adaptive)


# speedup vs baseline: 3.0487x; 1.0272x over previous
import functools

import jax
import jax.numpy as jnp
from jax import lax
from jax.experimental import pallas as pl
from jax.experimental.pallas import tpu as pltpu

B, T, N = 32, 12, 1024
INPUT_DIM = 3
IN_EMB, TOD_EMB, DOW_EMB, ADP_EMB = 24, 24, 24, 80
STEPS_PER_DAY = 288
OUT_DIM = IN_EMB + TOD_EMB + DOW_EMB + ADP_EMB
K_OH = 304
BB = 8


def _body(x_ref, w_ref, b_ref, tcat_ref, adp_ref, out_ref):
    M = BB * N
    xb = x_ref[:, 0].reshape(M, INPUT_DIM + 2)
    xi = xb[:, :INPUT_DIM]
    h = jnp.dot(xi, w_ref[...], preferred_element_type=jnp.float32) + b_ref[0]

    tod_idx = (xb[:, INPUT_DIM] * STEPS_PER_DAY).astype(jnp.int32)
    tod_idx = jnp.clip(tod_idx, 0, STEPS_PER_DAY - 1)
    dow_idx = xb[:, INPUT_DIM + 1].astype(jnp.int32)
    dow_idx = jnp.clip(dow_idx, 0, 6)

    lane = lax.broadcasted_iota(jnp.int32, (M, K_OH), 1)
    oh = ((lane == tod_idx[:, None])
          | (lane == dow_idx[:, None] + STEPS_PER_DAY)).astype(jnp.bfloat16)
    emb = jnp.dot(oh, tcat_ref[...], preferred_element_type=jnp.float32)

    h4 = h.reshape(BB, N, IN_EMB)
    e4 = emb.reshape(BB, N, OUT_DIM)
    for i in range(BB):
        out_ref[i, 0] = e4[i]
        out_ref[i, 0, :, 0:IN_EMB] = h4[i]
        out_ref[i, 0, :, IN_EMB + TOD_EMB + DOW_EMB:OUT_DIM] = adp_ref[0]


@functools.partial(jax.jit, static_argnames=("interpret",))
def kernel(x, W_proj, b_proj, tod_table, dow_table, adaptive, interpret=False):
    tcat = jnp.zeros((K_OH, OUT_DIM), jnp.bfloat16)
    tcat = tcat.at[:STEPS_PER_DAY, IN_EMB:IN_EMB + TOD_EMB].set(
        tod_table.astype(jnp.bfloat16))
    tcat = tcat.at[STEPS_PER_DAY:STEPS_PER_DAY + 7,
                   IN_EMB + TOD_EMB:IN_EMB + TOD_EMB + DOW_EMB].set(
        dow_table.astype(jnp.bfloat16))
    grid = (T, B // BB)
    return pl.pallas_call(
        _body,
        grid=grid,
        in_specs=[
            pl.BlockSpec((BB, 1, N, INPUT_DIM + 2), lambda t, b: (b, t, 0, 0)),
            pl.BlockSpec((INPUT_DIM, IN_EMB), lambda t, b: (0, 0)),
            pl.BlockSpec((1, IN_EMB), lambda t, b: (0, 0)),
            pl.BlockSpec((K_OH, OUT_DIM), lambda t, b: (0, 0)),
            pl.BlockSpec((1, N, ADP_EMB), lambda t, b: (t, 0, 0)),
        ],
        out_specs=pl.BlockSpec((BB, 1, N, OUT_DIM), lambda t, b: (b, t, 0, 0)),
        out_shape=jax.ShapeDtypeStruct((B, T, N, OUT_DIM), jnp.float32),
        compiler_params=pltpu.CompilerParams(
            dimension_semantics=("parallel", "parallel"),
        ),
        interpret=interpret,
    )(x, W_proj, b_proj.reshape(1, IN_EMB), tcat, adaptive)
